# baseline (speedup 1.0000x reference)
import functools

import jax
import jax.numpy as jnp
from jax import lax
from jax.experimental import pallas as pl
from jax.experimental.pallas import tpu as pltpu
from jax.experimental.pallas import tpu_sc as plsc

D_IN = 1024
D_H = 4096
D_OUT = 1024
B = 8192
SPARSITY = 0.5

N_SC = D_H * D_IN + D_H + D_OUT * D_H + D_OUT
J_RANK = int((1.0 - SPARSITY) * N_SC)

NC, NS, L = 2, 16, 16
NW = NC * NS
CHUNK = 16384
NW1 = D_H * D_IN
W_PER = NW1 // NW
WCH = W_PER // CHUNK
SB_N = 8192
SB_PER = SB_N // NW
NBINS = 65536


def _signed_key(bits):
    m = lax.shift_right_arithmetic(bits, 31)
    flip = lax.shift_right_logical(m, 1)
    return lax.bitwise_xor(bits, flip)


def _sortable_key(bits):
    m = lax.shift_right_arithmetic(bits, 31)
    flip = lax.bitwise_or(m, jnp.int32(-2147483648))
    return lax.bitwise_xor(bits, flip)


@functools.cache
def _sc_kernels():
    mesh = plsc.VectorSubcoreMesh(
        core_axis_name="c", subcore_axis_name="s",
        num_cores=NC, num_subcores=NS)

    def _zero_hist(hist_v):
        zeros = jnp.zeros((L,), jnp.int32)

        def zbody(i, _):
            hist_v[pl.ds(i * L, L)] = zeros
            return 0
        lax.fori_loop(0, NBINS // L, zbody, 0, unroll=8)

    def _streamed_hist(update, sw1_hbm, sw2_hbm, sb_hbm, out_hbm, wid,
                       buf0, buf1, bias_v, hist_v, sem0, sem1):
        bufs, sems = (buf0, buf1), (sem0, sem1)
        base = wid * W_PER
        srcs = ([(sw1_hbm, base + ci * CHUNK) for ci in range(WCH)]
                + [(sw2_hbm, base + ci * CHUNK) for ci in range(WCH)])

        def start(i):
            src, off = srcs[i]
            c = pltpu.make_async_copy(
                src.at[pl.ds(off, CHUNK)], bufs[i % 2], sems[i % 2])
            c.start()
            return c

        pending = start(0)
        bias_cp = pltpu.make_async_copy(
            sb_hbm.at[pl.ds(wid * SB_PER, SB_PER)], bias_v, sems[1])
        bias_cp.start()
        _zero_hist(hist_v)
        for i in range(len(srcs)):
            pending.wait()
            if i + 1 < len(srcs):
                pending = start(i + 1)
            lax.fori_loop(0, CHUNK // L,
                          functools.partial(update, bufs[i % 2]), 0, unroll=16)
        bias_cp.wait()
        lax.fori_loop(0, SB_PER // L,
                      functools.partial(update, bias_v), 0, unroll=8)
        pltpu.sync_copy(hist_v, out_hbm.at[wid])

    _scratch = [
        pltpu.VMEM((CHUNK,), jnp.int32),
        pltpu.VMEM((CHUNK,), jnp.int32),
        pltpu.VMEM((SB_PER,), jnp.int32),
        pltpu.VMEM((NBINS,), jnp.int32),
        pltpu.SemaphoreType.DMA,
        pltpu.SemaphoreType.DMA,
    ]

    @functools.partial(
        pl.kernel,
        compiler_params=pltpu.CompilerParams(needs_layout_passes=False),
        out_type=jax.ShapeDtypeStruct((NW, NBINS), jnp.int32),
        mesh=mesh,
        scratch_types=list(_scratch),
    )
    def _sc_hist_hi(sw1_hbm, sw2_hbm, sb_hbm, out_hbm,
                    buf0, buf1, bias_v, hist_v, sem0, sem1):
        wid = lax.axis_index("s") * NC + lax.axis_index("c")
        ones = jnp.ones((L,), jnp.int32)

        def update(buf, i, _):
            key = _sortable_key(buf[pl.ds(i * L, L)])
            b = lax.shift_right_logical(key, 16)
            plsc.addupdate_scatter(hist_v, [b], ones)
            return 0

        _streamed_hist(update, sw1_hbm, sw2_hbm, sb_hbm, out_hbm, wid,
                       buf0, buf1, bias_v, hist_v, sem0, sem1)

    @functools.partial(
        pl.kernel,
        compiler_params=pltpu.CompilerParams(needs_layout_passes=False),
        out_type=jax.ShapeDtypeStruct((NW, NBINS), jnp.int32),
        mesh=mesh,
        scratch_types=list(_scratch) + [pltpu.VMEM((128,), jnp.int32)],
    )
    def _sc_hist_lo(sw1_hbm, sw2_hbm, sb_hbm, bstar_hbm, out_hbm,
                    buf0, buf1, bias_v, hist_v, sem0, sem1, bvec_v):
        wid = lax.axis_index("s") * NC + lax.axis_index("c")
        pltpu.sync_copy(bstar_hbm, bvec_v)
        bv = bvec_v[pl.ds(0, L)]
        ones = jnp.ones((L,), jnp.int32)
        low_mask = jnp.full((L,), 0xFFFF, jnp.int32)

        def update(buf, i, _):
            key = _sortable_key(buf[pl.ds(i * L, L)])
            hi = lax.shift_right_logical(key, 16)
            lo = lax.bitwise_and(key, low_mask)
            plsc.addupdate_scatter(hist_v, [lo], ones, mask=hi == bv)
            return 0

        _streamed_hist(update, sw1_hbm, sw2_hbm, sb_hbm, out_hbm, wid,
                       buf0, buf1, bias_v, hist_v, sem0, sem1)

    return _sc_hist_hi, _sc_hist_lo


def _prefix_parts(h_i32):
    r0 = lax.broadcasted_iota(jnp.int32, (512, 512), 0)
    c0 = lax.broadcasted_iota(jnp.int32, (512, 512), 1)
    m_rows = (c0 < r0).astype(jnp.float32)
    r1 = lax.broadcasted_iota(jnp.int32, (128, 128), 0)
    c1 = lax.broadcasted_iota(jnp.int32, (128, 128), 1)
    m_cols = (r1 < c1).astype(jnp.float32)
    row_sums = jnp.sum(h_i32, axis=1, keepdims=True)

    def bit_slice(a_i32, k):
        return lax.bitwise_and(
            lax.shift_right_logical(a_i32, 8 * k), jnp.int32(255)
        ).astype(jnp.float32)

    row_pref = jnp.zeros((512, 1), jnp.float32)
    in_row = jnp.zeros((512, 128), jnp.float32)
    for k in range(3):
        scale = float(256 ** k)
        row_pref += scale * jnp.dot(
            m_rows, bit_slice(row_sums, k),
            preferred_element_type=jnp.float32)
        in_row += scale * jnp.dot(
            bit_slice(h_i32, k), m_cols,
            preferred_element_type=jnp.float32)
    excl = row_pref + in_row
    return excl, excl + h_i32.astype(jnp.float32)


def _tc_select_hi(hists_ref, out_ref):
    h = jnp.sum(hists_ref[...], axis=0)
    _, incl = _prefix_parts(h)
    ind = (incl <= float(J_RANK)).astype(jnp.float32)
    bstar = jnp.sum(ind)
    resid = float(J_RANK) - jnp.sum(h.astype(jnp.float32) * ind)
    out_ref[0, 0] = bstar.astype(jnp.int32)
    out_ref[0, 1] = resid.astype(jnp.int32)


def _tc_select_lo(hists_ref, br_ref, out_ref):
    h = jnp.sum(hists_ref[...], axis=0)
    _, incl = _prefix_parts(h)
    resid = br_ref[0, 1].astype(jnp.float32)
    ind = (incl <= resid).astype(jnp.float32)
    lostar = jnp.sum(ind).astype(jnp.int32)
    ustar = lax.bitwise_or(lax.shift_left(br_ref[0, 0], 16), lostar)
    out_ref[0, 0] = lax.bitwise_xor(ustar, jnp.int32(-2147483648))


BB = 512
CC2 = 512
NCH = D_H // CC2


def _tc_mlp_body(kt_ref, x_ref, w1_ref, s1_ref, w2t_ref, s2t_ref, b1_ref,
                 sb1_ref, b2_ref, sb2_ref, o_ref, w1m_s, w2m_s):
    b = pl.program_id(0)
    c = pl.program_id(1)
    kt = kt_ref[0, 0]

    @pl.when(b == 0)
    def _mask():
        zero = jnp.bfloat16(0)
        keep1 = _signed_key(s1_ref[...]) >= kt
        w1m_s[pl.ds(c * CC2, CC2), :] = jnp.where(
            keep1, w1_ref[...].astype(jnp.bfloat16), zero)
        keep2 = _signed_key(s2t_ref[...]) >= kt
        w2m_s[pl.ds(c * CC2, CC2), :] = jnp.where(
            keep2, w2t_ref[...].astype(jnp.bfloat16), zero)

    w1c = w1m_s[pl.ds(c * CC2, CC2), :]
    h32 = jax.lax.dot_general(x_ref[...], w1c,
                              (((1,), (1,)), ((), ())),
                              preferred_element_type=jnp.float32)
    b1m = jnp.where(_signed_key(sb1_ref[0]) >= kt, b1_ref[0], jnp.float32(0))
    h = jnp.maximum(h32 + b1m, 0.0).astype(jnp.bfloat16)
    w2c = w2m_s[pl.ds(c * CC2, CC2), :]
    part = jax.lax.dot_general(h, w2c,
                               (((1,), (0,)), ((), ())),
                               preferred_element_type=jnp.float32)

    @pl.when(c == 0)
    def _init():
        b2m = jnp.where(_signed_key(sb2_ref[...]) >= kt, b2_ref[...],
                        jnp.float32(0))
        o_ref[...] = part + b2m

    @pl.when(c != 0)
    def _acc():
        o_ref[...] += part


def kernel(x, W1, b1, W2, b2, s_W1, s_b1, s_W2, s_b2):
    i32 = jnp.int32
    sw1_bits = lax.bitcast_convert_type(s_W1, i32)
    sw2_bits = lax.bitcast_convert_type(s_W2, i32)
    sb_bits = lax.bitcast_convert_type(jnp.concatenate([
        s_b1, s_b2, jnp.full((SB_N - D_H - D_OUT,), jnp.inf, jnp.float32)
    ]), i32)
    sw1_flat = sw1_bits.reshape(-1)
    sw2_flat = sw2_bits.reshape(-1)

    sc_hist_hi, sc_hist_lo = _sc_kernels()
    hist_hi = sc_hist_hi(sw1_flat, sw2_flat, sb_bits)
    br = pl.pallas_call(
        _tc_select_hi,
        grid=(),
        in_specs=[pl.BlockSpec(memory_space=pltpu.VMEM)],
        out_specs=pl.BlockSpec(memory_space=pltpu.SMEM),
        out_shape=jax.ShapeDtypeStruct((1, 2), i32),
    )(hist_hi.reshape(NW, 512, 128))
    bstar_vec = jnp.broadcast_to(br[0, 0], (128,))
    hist_lo = sc_hist_lo(sw1_flat, sw2_flat, sb_bits, bstar_vec)
    kt = pl.pallas_call(
        _tc_select_lo,
        grid=(),
        in_specs=[pl.BlockSpec(memory_space=pltpu.VMEM),
                  pl.BlockSpec(memory_space=pltpu.SMEM)],
        out_specs=pl.BlockSpec(memory_space=pltpu.SMEM),
        out_shape=jax.ShapeDtypeStruct((1, 1), i32),
    )(hist_lo.reshape(NW, 512, 128), br)

    x_bf = x.astype(jnp.bfloat16)
    w2t = W2.T
    sw2t = sw2_bits.T

    def _w_idx(b, c):
        return (jnp.where(b == 0, c, 0), 0)

    out = pl.pallas_call(
        _tc_mlp_body,
        grid=(B // BB, NCH),
        in_specs=[
            pl.BlockSpec(memory_space=pltpu.SMEM),
            pl.BlockSpec((BB, D_IN), lambda b, c: (b, 0)),
            pl.BlockSpec((CC2, D_IN), _w_idx),
            pl.BlockSpec((CC2, D_IN), _w_idx),
            pl.BlockSpec((CC2, D_OUT), _w_idx),
            pl.BlockSpec((CC2, D_OUT), _w_idx),
            pl.BlockSpec((1, 1, CC2), lambda b, c: (c, 0, 0)),
            pl.BlockSpec((1, 1, CC2), lambda b, c: (c, 0, 0)),
            pl.BlockSpec((1, D_OUT), lambda b, c: (0, 0)),
            pl.BlockSpec((1, D_OUT), lambda b, c: (0, 0)),
        ],
        out_specs=pl.BlockSpec((BB, D_OUT), lambda b, c: (b, 0)),
        out_shape=jax.ShapeDtypeStruct((B, D_OUT), jnp.float32),
        scratch_shapes=[
            pltpu.VMEM((D_H, D_IN), jnp.bfloat16),
            pltpu.VMEM((D_H, D_OUT), jnp.bfloat16),
        ],
    )(kt, x_bf, W1, sw1_bits, w2t, sw2t,
      b1.reshape(NCH, 1, CC2),
      lax.bitcast_convert_type(s_b1, i32).reshape(NCH, 1, CC2),
      b2.reshape(1, D_OUT),
      lax.bitcast_convert_type(s_b2, i32).reshape(1, D_OUT))
    return out


# speedup vs baseline: 1.1608x; 1.1608x over previous
import functools

import jax
import jax.numpy as jnp
from jax import lax
from jax.experimental import pallas as pl
from jax.experimental.pallas import tpu as pltpu
from jax.experimental.pallas import tpu_sc as plsc

D_IN = 1024
D_H = 4096
D_OUT = 1024
B = 8192
SPARSITY = 0.5

N_SC = D_H * D_IN + D_H + D_OUT * D_H + D_OUT
J_RANK = int((1.0 - SPARSITY) * N_SC)

NC, NS, L = 2, 16, 16
NW = NC * NS
SB_N = 8192
SB_PER = SB_N // NW
NBINS = 65536

W1_ROWS_PER_W = D_H // NW
W2_ROWS_PER_W = D_OUT // NW
C1_SHAPE = (8, D_IN)
C2_SHAPE = (8, D_H // 2)
N_C1 = W1_ROWS_PER_W // 8
N_C2 = (W2_ROWS_PER_W // 8) * 2


def _signed_key(bits):
    m = lax.shift_right_arithmetic(bits, 31)
    flip = lax.shift_right_logical(m, 1)
    return lax.bitwise_xor(bits, flip)


def _sortable_key(bits):
    m = lax.shift_right_arithmetic(bits, 31)
    flip = lax.bitwise_or(m, jnp.int32(-2147483648))
    return lax.bitwise_xor(bits, flip)


@functools.cache
def _sc_kernels():
    mesh = plsc.VectorSubcoreMesh(
        core_axis_name="c", subcore_axis_name="s",
        num_cores=NC, num_subcores=NS)

    def _zero_hist(hist_v):
        zeros = jnp.zeros((L,), jnp.int32)

        def zbody(i, _):
            hist_v[lax.shift_right_logical(i, 3),
                   pl.ds(lax.bitwise_and(i, 7) * L, L)] = zeros
            return 0
        lax.fori_loop(0, NBINS // L, zbody, 0, unroll=8)

    def _streamed_hist(update, sw1_hbm, sw2_hbm, sb_hbm, out_hbm, wid,
                       a0, a1, b0, b1, bias_v, hist_v, sem0, sem1):
        base1 = wid * W1_ROWS_PER_W
        base2 = wid * W2_ROWS_PER_W
        plan = ([(lambda ci=ci: sw1_hbm.at[pl.ds(base1 + ci * 8, 8)],
                  (a0, a1), C1_SHAPE) for ci in range(N_C1)]
                + [(lambda ci=ci: sw2_hbm.at[
                      pl.ds(base2 + (ci // 2) * 8, 8),
                      pl.ds((ci % 2) * (D_H // 2), D_H // 2)],
                    (b0, b1), C2_SHAPE) for ci in range(N_C2)])
        sems = (sem0, sem1)

        def start(i):
            mk, bufs, _ = plan[i]
            c = pltpu.make_async_copy(mk(), bufs[i % 2], sems[i % 2])
            c.start()
            return c

        pending = start(0)
        bias_cp = pltpu.make_async_copy(
            sb_hbm.at[pl.ds(wid * SB_PER, SB_PER)], bias_v, sems[1])
        bias_cp.start()
        _zero_hist(hist_v)
        for i in range(len(plan)):
            _, bufs, shape = plan[i]
            buf = bufs[i % 2]
            vregs_per_row = shape[1] // L
            shift = vregs_per_row.bit_length() - 1
            pending.wait()
            if i + 1 < len(plan):
                pending = start(i + 1)

            def vbody(k, _, buf=buf, shift=shift):
                r = lax.shift_right_logical(k, shift)
                cstart = lax.bitwise_and(k, (1 << shift) - 1) * L
                update(buf[r, pl.ds(cstart, L)])
                return 0
            lax.fori_loop(0, shape[0] * vregs_per_row, vbody, 0, unroll=16)
        bias_cp.wait()

        def bbody(k, _):
            update(bias_v[pl.ds(k * L, L)])
            return 0
        lax.fori_loop(0, SB_PER // L, bbody, 0, unroll=8)
        pltpu.sync_copy(hist_v, out_hbm.at[wid])

    _scratch = [
        pltpu.VMEM(C1_SHAPE, jnp.int32),
        pltpu.VMEM(C1_SHAPE, jnp.int32),
        pltpu.VMEM(C2_SHAPE, jnp.int32),
        pltpu.VMEM(C2_SHAPE, jnp.int32),
        pltpu.VMEM((SB_PER,), jnp.int32),
        pltpu.VMEM((512, 128), jnp.int32),
        pltpu.SemaphoreType.DMA,
        pltpu.SemaphoreType.DMA,
    ]
    _params = pltpu.CompilerParams(
        needs_layout_passes=False, use_tc_tiling_on_sc=True)

    @functools.partial(
        pl.kernel,
        compiler_params=_params,
        out_type=jax.ShapeDtypeStruct((NW, 512, 128), jnp.int32),
        mesh=mesh,
        scratch_types=list(_scratch),
    )
    def _sc_hist_hi(sw1_hbm, sw2_hbm, sb_hbm, out_hbm,
                    a0, a1, b0, b1, bias_v, hist_v, sem0, sem1):
        wid = lax.axis_index("s") * NC + lax.axis_index("c")
        ones = jnp.ones((L,), jnp.int32)

        def update(bits):
            key = _sortable_key(bits)
            b = lax.shift_right_logical(key, 16)
            plsc.addupdate_scatter(
                hist_v,
                [lax.shift_right_logical(b, 7), lax.bitwise_and(b, 127)],
                ones)

        _streamed_hist(update, sw1_hbm, sw2_hbm, sb_hbm, out_hbm, wid,
                       a0, a1, b0, b1, bias_v, hist_v, sem0, sem1)

    @functools.partial(
        pl.kernel,
        compiler_params=_params,
        out_type=jax.ShapeDtypeStruct((NW, 512, 128), jnp.int32),
        mesh=mesh,
        scratch_types=list(_scratch) + [pltpu.VMEM((128,), jnp.int32)],
    )
    def _sc_hist_lo(sw1_hbm, sw2_hbm, sb_hbm, bstar_hbm, out_hbm,
                    a0, a1, b0, b1, bias_v, hist_v, sem0, sem1, bvec_v):
        wid = lax.axis_index("s") * NC + lax.axis_index("c")
        pltpu.sync_copy(bstar_hbm, bvec_v)
        bv = bvec_v[pl.ds(0, L)]
        ones = jnp.ones((L,), jnp.int32)
        low_mask = jnp.full((L,), 0xFFFF, jnp.int32)

        def update(bits):
            key = _sortable_key(bits)
            hi = lax.shift_right_logical(key, 16)
            lo = lax.bitwise_and(key, low_mask)
            plsc.addupdate_scatter(
                hist_v,
                [lax.shift_right_logical(lo, 7), lax.bitwise_and(lo, 127)],
                ones, mask=hi == bv)

        _streamed_hist(update, sw1_hbm, sw2_hbm, sb_hbm, out_hbm, wid,
                       a0, a1, b0, b1, bias_v, hist_v, sem0, sem1)

    return _sc_hist_hi, _sc_hist_lo


def _prefix_parts(h_i32):
    r0 = lax.broadcasted_iota(jnp.int32, (512, 512), 0)
    c0 = lax.broadcasted_iota(jnp.int32, (512, 512), 1)
    m_rows = (c0 < r0).astype(jnp.float32)
    r1 = lax.broadcasted_iota(jnp.int32, (128, 128), 0)
    c1 = lax.broadcasted_iota(jnp.int32, (128, 128), 1)
    m_cols = (r1 < c1).astype(jnp.float32)
    row_sums = jnp.sum(h_i32, axis=1, keepdims=True)

    def bit_slice(a_i32, k):
        return lax.bitwise_and(
            lax.shift_right_logical(a_i32, 8 * k), jnp.int32(255)
        ).astype(jnp.float32)

    row_pref = jnp.zeros((512, 1), jnp.float32)
    in_row = jnp.zeros((512, 128), jnp.float32)
    for k in range(3):
        scale = float(256 ** k)
        row_pref += scale * jnp.dot(
            m_rows, bit_slice(row_sums, k),
            preferred_element_type=jnp.float32)
        in_row += scale * jnp.dot(
            bit_slice(h_i32, k), m_cols,
            preferred_element_type=jnp.float32)
    excl = row_pref + in_row
    return excl, excl + h_i32.astype(jnp.float32)


def _tc_select_hi(hists_ref, out_ref):
    h = jnp.sum(hists_ref[...], axis=0)
    _, incl = _prefix_parts(h)
    ind = (incl <= float(J_RANK)).astype(jnp.float32)
    bstar = jnp.sum(ind)
    resid = float(J_RANK) - jnp.sum(h.astype(jnp.float32) * ind)
    out_ref[0, 0] = bstar.astype(jnp.int32)
    out_ref[0, 1] = resid.astype(jnp.int32)


def _tc_select_lo(hists_ref, br_ref, out_ref):
    h = jnp.sum(hists_ref[...], axis=0)
    _, incl = _prefix_parts(h)
    resid = br_ref[0, 1].astype(jnp.float32)
    ind = (incl <= resid).astype(jnp.float32)
    lostar = jnp.sum(ind).astype(jnp.int32)
    ustar = lax.bitwise_or(lax.shift_left(br_ref[0, 0], 16), lostar)
    out_ref[0, 0] = lax.bitwise_xor(ustar, jnp.int32(-2147483648))


def _tc_mask_body(kt_ref, w1_ref, s1_ref, w2_ref, s2_ref, o1_ref, o2_ref):
    kt = kt_ref[0, 0]
    zero = jnp.bfloat16(0)
    keep1 = _signed_key(lax.bitcast_convert_type(s1_ref[...], jnp.int32)) >= kt
    o1_ref[...] = jnp.where(keep1, w1_ref[...].astype(jnp.bfloat16), zero)
    keep2 = _signed_key(lax.bitcast_convert_type(s2_ref[...], jnp.int32)) >= kt
    o2_ref[...] = jnp.where(keep2, w2_ref[...].astype(jnp.bfloat16), zero)


def _tc_mlp_body(kt_ref, x_ref, w1_ref, b1_ref, sb1_ref, w2_ref, b2_ref,
                 sb2_ref, o_ref):
    c = pl.program_id(1)
    kt = kt_ref[0, 0]
    b1m = jnp.where(
        _signed_key(lax.bitcast_convert_type(sb1_ref[0], jnp.int32)) >= kt,
        b1_ref[0], jnp.float32(0))
    h32 = jax.lax.dot_general(x_ref[...], w1_ref[...],
                              (((1,), (1,)), ((), ())),
                              preferred_element_type=jnp.float32)
    h = jnp.maximum(h32 + b1m, 0.0).astype(jnp.bfloat16)
    part = jax.lax.dot_general(h, w2_ref[...],
                               (((1,), (1,)), ((), ())),
                               preferred_element_type=jnp.float32)

    @pl.when(c == 0)
    def _init():
        b2m = jnp.where(
            _signed_key(lax.bitcast_convert_type(sb2_ref[...], jnp.int32))
            >= kt, b2_ref[...], jnp.float32(0))
        o_ref[...] = part + b2m

    @pl.when(c != 0)
    def _acc():
        o_ref[...] += part


def kernel(x, W1, b1, W2, b2, s_W1, s_b1, s_W2, s_b2):
    i32 = jnp.int32
    sw1_bits = lax.bitcast_convert_type(s_W1, i32)
    sw2_bits = lax.bitcast_convert_type(s_W2, i32)
    sb_bits = lax.bitcast_convert_type(jnp.concatenate([
        s_b1, s_b2, jnp.full((SB_N - D_H - D_OUT,), jnp.inf, jnp.float32)
    ]), i32)

    sc_hist_hi, sc_hist_lo = _sc_kernels()
    hist_hi = sc_hist_hi(sw1_bits, sw2_bits, sb_bits)
    br = pl.pallas_call(
        _tc_select_hi,
        grid=(),
        in_specs=[pl.BlockSpec(memory_space=pltpu.VMEM)],
        out_specs=pl.BlockSpec(memory_space=pltpu.SMEM),
        out_shape=jax.ShapeDtypeStruct((1, 2), i32),
    )(hist_hi)
    bstar_vec = jnp.broadcast_to(br[0, 0], (128,))
    hist_lo = sc_hist_lo(sw1_bits, sw2_bits, sb_bits, bstar_vec)
    kt = pl.pallas_call(
        _tc_select_lo,
        grid=(),
        in_specs=[pl.BlockSpec(memory_space=pltpu.VMEM),
                  pl.BlockSpec(memory_space=pltpu.SMEM)],
        out_specs=pl.BlockSpec(memory_space=pltpu.SMEM),
        out_shape=jax.ShapeDtypeStruct((1, 1), i32),
    )(hist_lo, br)

    MB = 512
    W1m, W2m_rs = pl.pallas_call(
        _tc_mask_body,
        grid=(D_H // MB,),
        in_specs=[
            pl.BlockSpec(memory_space=pltpu.SMEM),
            pl.BlockSpec((MB, D_IN), lambda i: (i, 0)),
            pl.BlockSpec((MB, D_IN), lambda i: (i, 0)),
            pl.BlockSpec((MB, D_IN), lambda i: (i, 0)),
            pl.BlockSpec((MB, D_IN), lambda i: (i, 0)),
        ],
        out_specs=[
            pl.BlockSpec((MB, D_IN), lambda i: (i, 0)),
            pl.BlockSpec((MB, D_IN), lambda i: (i, 0)),
        ],
        out_shape=[
            jax.ShapeDtypeStruct((D_H, D_IN), jnp.bfloat16),
            jax.ShapeDtypeStruct((D_H, D_IN), jnp.bfloat16),
        ],
    )(kt, W1, s_W1, W2.reshape(D_H, D_IN), s_W2.reshape(D_H, D_IN))
    W2m = W2m_rs.reshape(D_OUT, D_H)

    x_bf = x.astype(jnp.bfloat16)
    BB, CC = 512, 1024
    out = pl.pallas_call(
        _tc_mlp_body,
        grid=(B // BB, D_H // CC),
        in_specs=[
            pl.BlockSpec(memory_space=pltpu.SMEM),
            pl.BlockSpec((BB, D_IN), lambda b, c: (b, 0)),
            pl.BlockSpec((CC, D_IN), lambda b, c: (c, 0)),
            pl.BlockSpec((1, 1, CC), lambda b, c: (c, 0, 0)),
            pl.BlockSpec((1, 1, CC), lambda b, c: (c, 0, 0)),
            pl.BlockSpec((D_OUT, CC), lambda b, c: (0, c)),
            pl.BlockSpec((1, D_OUT), lambda b, c: (0, 0)),
            pl.BlockSpec((1, D_OUT), lambda b, c: (0, 0)),
        ],
        out_specs=pl.BlockSpec((BB, D_OUT), lambda b, c: (b, 0)),
        out_shape=jax.ShapeDtypeStruct((B, D_OUT), jnp.float32),
    )(kt, x_bf, W1m, b1.reshape(D_H // CC, 1, CC),
      s_b1.reshape(D_H // CC, 1, CC),
      W2m, b2.reshape(1, D_OUT),
      s_b2.reshape(1, D_OUT))
    return out
